# baseline (speedup 1.0000x reference)
import functools

import jax
import jax.numpy as jnp
from jax import lax
from jax.experimental import pallas as pl
from jax.experimental.pallas import tpu as pltpu
from jax.experimental.pallas import tpu_sc as plsc

_C = 100
_CP = 128
_A = 128
_L = 16
_NC = 2
_NS = 16
_NW = _NC * _NS
_B = 80
_NBUF = 5


def _sc_partials(features, labels):
  n = features.shape[0]
  rows_per_w = n // _NW
  chunk = _B
  nch = rows_per_w // chunk
  assert nch % _NBUF == 0

  mesh = plsc.VectorSubcoreMesh(core_axis_name="c", subcore_axis_name="s")

  @functools.partial(
      pl.kernel,
      out_type=[
          jax.ShapeDtypeStruct((_NC, _CP, _A), jnp.float32),
          jax.ShapeDtypeStruct((_NW, _CP * _L), jnp.float32),
      ],
      mesh=mesh,
      compiler_params=pltpu.CompilerParams(needs_layout_passes=False),
      scratch_types=[
          pltpu.VMEM((rows_per_w + _L,), jnp.int32),
          pltpu.VMEM((nch, _B), jnp.int32),
          [pltpu.VMEM((chunk, _A), jnp.float32) for _ in range(_NBUF)],
          pltpu.VMEM((_CP, _A), jnp.float32),
          pltpu.VMEM_SHARED((_CP, _A), jnp.float32),
          pltpu.VMEM((_CP * _L,), jnp.float32),
          [pltpu.SemaphoreType.DMA for _ in range(_NBUF)],
          pltpu.SemaphoreType.DMA,
          pltpu.SemaphoreType.DMA,
      ],
  )
  def k(feat_hbm, lab_hbm, out_sums, out_cnt,
        lab_v, lab2d, bufs, zbuf, acc_sh, cnt, sems, lsem, l3sem):
    cid = lax.axis_index("c")
    sid = lax.axis_index("s")
    wid = cid * _NS + sid
    base = wid * rows_per_w

    zeros = jnp.zeros((_L,), jnp.float32)

    def lab2d_issue(g, _):
      pltpu.async_copy(lab_hbm.at[pl.ds(base + g * _B, _B)],
                       lab2d.at[g], l3sem)
      return 0
    lax.fori_loop(0, nch, lab2d_issue, 0)
    lab_cp = pltpu.async_copy(lab_hbm.at[pl.ds(base, rows_per_w)],
                              lab_v.at[pl.ds(0, rows_per_w)], lsem)

    def start_dma(g, buf, sem):
      return pltpu.async_copy(
          feat_hbm.at[pl.ds(base + g * chunk, chunk)], buf, sem)

    def wait_dma(buf, sem):
      pltpu.make_async_copy(feat_hbm.at[pl.ds(base, chunk)], buf, sem).wait()

    for b in range(_NBUF):
      start_dma(b, bufs[b], sems[b])

    @pl.when(sid == 0)
    def _():
      def zero_acc(i, _):
        for j in range(_A // _L):
          zbuf[i, pl.ds(j * _L, _L)] = zeros
        return 0
      lax.fori_loop(0, _CP, zero_acc, 0)
      pltpu.sync_copy(zbuf, acc_sh)
    plsc.subcore_barrier()

    def zero_cnt(i, _):
      cnt[pl.ds(i * _L, _L)] = zeros
      return 0
    lax.fori_loop(0, _CP, zero_cnt, 0)

    lane = lax.iota(jnp.int32, _L)
    ones = jnp.ones((_L,), jnp.float32)

    lab_cp.wait()

    def cnt_body(g, _):
      lab16 = lab_v[pl.ds(g * _L, _L)]
      plsc.addupdate_scatter(cnt, [lab16 * _L + lane], ones)
      return 0
    lax.fori_loop(0, rows_per_w // _L, cnt_body, 0)

    def lab2d_drain(g, _):
      pltpu.make_async_copy(lab_hbm.at[pl.ds(base, _B)],
                            lab2d.at[0], l3sem).wait()
      return 0
    lax.fori_loop(0, nch, lab2d_drain, 0)

    def chunk_body(h, _):
      for b in range(_NBUF):
        g = h * _NBUF + b
        wait_dma(bufs[b], sems[b])
        pltpu.sync_copy(bufs[b], acc_sh.at[lab2d.at[g]], add=True)

        @pl.when(g + _NBUF < nch)
        def _():
          start_dma(g + _NBUF, bufs[b], sems[b])
      return 0
    lax.fori_loop(0, nch // _NBUF, chunk_body, 0)

    plsc.subcore_barrier()
    @pl.when(sid == 0)
    def _():
      pltpu.sync_copy(acc_sh, out_sums.at[cid])
    pltpu.sync_copy(cnt, out_cnt.at[wid])

  return k(features, labels)


def _combine_kernel(sums_ref, cnt_ref, out_ref):
  s = sums_ref[0] + sums_ref[1]
  c = jnp.sum(cnt_ref[...], axis=(0, 2))
  denom = jnp.where(c == 0.0, 1.0, c)
  out_ref[...] = (s / denom[:, None])[:_C]


def _combine(partial_sums, partial_cnt):
  return pl.pallas_call(
      _combine_kernel,
      out_shape=jax.ShapeDtypeStruct((_C, _A), jnp.float32),
  )(partial_sums, partial_cnt)


@jax.jit
def kernel(features, labels):
  partial_sums, partial_cnt = _sc_partials(features, labels)
  partial_cnt = partial_cnt.reshape(_NW, _CP, _L)
  avg = _combine(partial_sums, partial_cnt)
  return lax.stop_gradient(avg)


# speedup vs baseline: 1.0048x; 1.0048x over previous
import functools

import jax
import jax.numpy as jnp
from jax import lax
from jax.experimental import pallas as pl
from jax.experimental.pallas import tpu as pltpu
from jax.experimental.pallas import tpu_sc as plsc

_C = 100
_CP = 128
_A = 128
_L = 16
_NC = 2
_NS = 16
_NW = _NC * _NS
_B = 80
_NBUF = 5


def _sc_partials(features, labels, labels3d):
  n = features.shape[0]
  rows_per_w = n // _NW
  chunk = _B
  nch = rows_per_w // chunk
  assert nch % _NBUF == 0

  mesh = plsc.VectorSubcoreMesh(core_axis_name="c", subcore_axis_name="s")

  @functools.partial(
      pl.kernel,
      out_type=[
          jax.ShapeDtypeStruct((_NC, _CP, _A), jnp.float32),
          jax.ShapeDtypeStruct((_NW, _CP * _L), jnp.float32),
      ],
      mesh=mesh,
      compiler_params=pltpu.CompilerParams(needs_layout_passes=False),
      scratch_types=[
          pltpu.VMEM((rows_per_w + _L,), jnp.int32),
          pltpu.VMEM((nch, _B), jnp.int32),
          [pltpu.VMEM((chunk, _A), jnp.float32) for _ in range(_NBUF)],
          pltpu.VMEM((_CP, _A), jnp.float32),
          pltpu.VMEM_SHARED((_CP, _A), jnp.float32),
          pltpu.VMEM((_CP * _L,), jnp.float32),
          [pltpu.SemaphoreType.DMA for _ in range(_NBUF)],
          pltpu.SemaphoreType.DMA,
          pltpu.SemaphoreType.DMA,
      ],
  )
  def k(feat_hbm, lab_hbm, lab3d_hbm, out_sums, out_cnt,
        lab_v, lab2d, bufs, zbuf, acc_sh, cnt, sems, lsem, l3sem):
    cid = lax.axis_index("c")
    sid = lax.axis_index("s")
    wid = cid * _NS + sid
    base = wid * rows_per_w

    zeros = jnp.zeros((_L,), jnp.float32)

    lab2d_cp = pltpu.async_copy(lab3d_hbm.at[wid], lab2d, l3sem)
    lab_cp = pltpu.async_copy(lab_hbm.at[pl.ds(base, rows_per_w)],
                              lab_v.at[pl.ds(0, rows_per_w)], lsem)

    def start_dma(g, buf, sem):
      return pltpu.async_copy(
          feat_hbm.at[pl.ds(base + g * chunk, chunk)], buf, sem)

    def wait_dma(buf, sem):
      pltpu.make_async_copy(feat_hbm.at[pl.ds(base, chunk)], buf, sem).wait()

    for b in range(_NBUF):
      start_dma(b, bufs[b], sems[b])

    @pl.when(sid == 0)
    def _():
      def zero_acc(i, _):
        for j in range(_A // _L):
          zbuf[i, pl.ds(j * _L, _L)] = zeros
        return 0
      lax.fori_loop(0, _CP, zero_acc, 0)
      pltpu.sync_copy(zbuf, acc_sh)
    plsc.subcore_barrier()

    def zero_cnt(i, _):
      cnt[pl.ds(i * _L, _L)] = zeros
      return 0
    lax.fori_loop(0, _CP, zero_cnt, 0)

    lane = lax.iota(jnp.int32, _L)
    ones = jnp.ones((_L,), jnp.float32)

    lab_cp.wait()

    def cnt_body(g, _):
      lab16 = lab_v[pl.ds(g * _L, _L)]
      plsc.addupdate_scatter(cnt, [lab16 * _L + lane], ones)
      return 0
    lax.fori_loop(0, rows_per_w // _L, cnt_body, 0)

    lab2d_cp.wait()

    def chunk_body(h, _):
      for b in range(_NBUF):
        g = h * _NBUF + b
        wait_dma(bufs[b], sems[b])
        pltpu.sync_copy(bufs[b], acc_sh.at[lab2d.at[g]], add=True)

        @pl.when(g + _NBUF < nch)
        def _():
          start_dma(g + _NBUF, bufs[b], sems[b])
      return 0
    lax.fori_loop(0, nch // _NBUF, chunk_body, 0)

    plsc.subcore_barrier()
    @pl.when(sid == 0)
    def _():
      pltpu.sync_copy(acc_sh, out_sums.at[cid])
    pltpu.sync_copy(cnt, out_cnt.at[wid])

  return k(features, labels, labels3d)


def _combine_kernel(sums_ref, cnt_ref, out_ref):
  s = sums_ref[0] + sums_ref[1]
  c = jnp.sum(cnt_ref[...], axis=(0, 2))
  denom = jnp.where(c == 0.0, 1.0, c)
  out_ref[...] = (s / denom[:, None])[:_C]


def _combine(partial_sums, partial_cnt):
  return pl.pallas_call(
      _combine_kernel,
      out_shape=jax.ShapeDtypeStruct((_C, _A), jnp.float32),
  )(partial_sums, partial_cnt)


@jax.jit
def kernel(features, labels):
  labels3d = labels.reshape(_NW, -1, _B)
  partial_sums, partial_cnt = _sc_partials(features, labels, labels3d)
  partial_cnt = partial_cnt.reshape(_NW, _CP, _L)
  avg = _combine(partial_sums, partial_cnt)
  return lax.stop_gradient(avg)
